# baseline (speedup 1.0000x reference)
import functools

import jax
import jax.numpy as jnp
from jax.experimental import pallas as pl
from jax.experimental.pallas import tpu as pltpu

B = 512
D = 512
C = 3
DELTA = 1.5


def _fused_kernel(tf_hbm, af_hbm, vf_hbm, tl_ref, al_ref, vl_ref, fr_hbm,
                  lc_ref, adj_hbm, nf_hbm,
                  ft_v, fa_v, fv_v, fr_v, adj_v,
                  sem_ft, sem_fa, sem_fv, sem_fr, sem_nf0, sem_nf1, sem_adj):
    cp_ft = pltpu.make_async_copy(tf_hbm, ft_v, sem_ft)
    cp_fa = pltpu.make_async_copy(af_hbm, fa_v, sem_fa)
    cp_fv = pltpu.make_async_copy(vf_hbm, fv_v, sem_fv)
    cp_fr = pltpu.make_async_copy(fr_hbm, fr_v, sem_fr)
    cp_nf0 = pltpu.make_async_copy(fr_hbm, nf_hbm.at[pl.ds(0, B), :], sem_nf0)
    cp_nf1 = pltpu.make_async_copy(lc_ref, nf_hbm.at[pl.ds(B, C), :], sem_nf1)
    cp_ft.start()
    cp_fa.start()
    cp_fv.start()
    cp_fr.start()
    cp_nf0.start()
    cp_nf1.start()

    s = 0.5 / (3.0 * DELTA)

    mods = []
    for l_ref in (tl_ref, al_ref, vl_ref):
        labels = l_ref[:]
        pred = jnp.argmax(labels, axis=1)
        onehot = (pred[:, None] == jnp.arange(C)[None, :]).astype(jnp.float32)
        logp = jnp.log(labels)
        h = jnp.sum(labels * logp, axis=1, keepdims=True)
        ones = jnp.ones((B, 1), dtype=jnp.float32)
        U = jnp.concatenate([labels, logp, h, ones], axis=1)
        V = jnp.concatenate([s * logp, s * labels, -s * ones,
                             1.0 / 3.0 - s * h], axis=1)
        T = jnp.dot(U, V.T, preferred_element_type=jnp.float32)
        mods.append((T, onehot))

    wacc = jnp.zeros((B, B), dtype=jnp.float32)
    centers_sum = jnp.zeros((C, D), dtype=jnp.float32)
    for (T, onehot), cp, f_v in zip(mods, (cp_ft, cp_fa, cp_fv),
                                    (ft_v, fa_v, fv_v)):
        cp.wait()
        feats = f_v[:]

        counts = jnp.sum(onehot, axis=0)
        centers = jnp.dot(onehot.T, feats,
                          preferred_element_type=jnp.float32)
        centers = centers / jnp.maximum(counts, 1.0)[:, None]
        centers_sum = centers_sum + centers

        f2 = jnp.sum(feats * feats, axis=1)
        c2 = jnp.sum(centers * centers, axis=1)
        G = jnp.dot(feats, centers.T,
                    preferred_element_type=jnp.float32)
        g = jnp.exp(-0.5 * (f2[:, None] + c2[None, :] - 2.0 * G))

        dij = jnp.dot(g, onehot.T, preferred_element_type=jnp.float32)
        dji = jnp.dot(onehot, g.T, preferred_element_type=jnp.float32)

        wacc = wacc + jnp.maximum(T, 0.0) * dij * dji

    ri = jax.lax.broadcasted_iota(jnp.int32, (B, B), 0)
    ci = jax.lax.broadcasted_iota(jnp.int32, (B, B), 1)
    w = jnp.where(ri == ci, 0.0, wacc)

    cp_fr.wait()
    fused = fr_v[:]
    lcent = lc_ref[:]
    logits = jnp.dot(fused, lcent.T,
                     preferred_element_type=jnp.float32)
    pred_f = jnp.argmax(logits, axis=1)
    onehot_f = (pred_f[:, None] == jnp.arange(C)[None, :]).astype(jnp.float32)

    avg_c = centers_sum * (1.0 / 3.0)
    fu2 = jnp.sum(fused * fused, axis=1)
    a2 = jnp.sum(avg_c * avg_c, axis=1)
    Gf = jnp.dot(fused, avg_c.T,
                 preferred_element_type=jnp.float32)
    d2f = fu2 + jnp.sum(onehot_f * (a2[None, :] - 2.0 * Gf), axis=1)
    wf = jnp.exp(-0.5 * d2f)

    R = wf[:, None] * onehot_f

    adj_v[0:B, 0:B] = w
    adj_v[0:B, B:B + C] = R
    adj_v[B:B + C, 0:B] = R.T
    adj_v[B:B + C, B:B + C] = jnp.zeros((C, C), dtype=jnp.float32)

    cp_adj = pltpu.make_async_copy(adj_v, adj_hbm, sem_adj)
    cp_adj.start()
    cp_adj.wait()
    cp_nf0.wait()
    cp_nf1.wait()


@functools.partial(jax.jit)
def kernel(text_features, audio_features, vision_features, text_labels,
           audio_labels, vision_labels, fused_representations,
           learnable_class_centers):
    n = B + C
    any_spec = pl.BlockSpec(memory_space=pl.ANY)
    vmem_spec = pl.BlockSpec(memory_space=pltpu.VMEM)
    adj, node_features = pl.pallas_call(
        _fused_kernel,
        in_specs=[any_spec, any_spec, any_spec,
                  vmem_spec, vmem_spec, vmem_spec,
                  any_spec, vmem_spec],
        out_specs=(any_spec, any_spec),
        out_shape=(
            jax.ShapeDtypeStruct((n, n), jnp.float32),
            jax.ShapeDtypeStruct((n, D), jnp.float32),
        ),
        scratch_shapes=[
            pltpu.VMEM((B, D), jnp.float32),
            pltpu.VMEM((B, D), jnp.float32),
            pltpu.VMEM((B, D), jnp.float32),
            pltpu.VMEM((B, D), jnp.float32),
            pltpu.VMEM((n, n), jnp.float32),
            pltpu.SemaphoreType.DMA,
            pltpu.SemaphoreType.DMA,
            pltpu.SemaphoreType.DMA,
            pltpu.SemaphoreType.DMA,
            pltpu.SemaphoreType.DMA,
            pltpu.SemaphoreType.DMA,
            pltpu.SemaphoreType.DMA,
        ],
    )(text_features, audio_features, vision_features,
      text_labels, audio_labels, vision_labels,
      fused_representations, learnable_class_centers)
    return adj, node_features


# speedup vs baseline: 2.4457x; 2.4457x over previous
import functools

import jax
import jax.numpy as jnp
from jax.experimental import pallas as pl
from jax.experimental.pallas import tpu as pltpu

B = 512
D = 512
C = 3
DELTA = 1.5


def _fused_kernel(tf_ref, af_ref, vf_ref, tl_hbm, al_hbm, vl_hbm, fr_ref,
                  lc_ref, adj_ref, nf_ref,
                  tl_v, al_v, vl_v, sem_tl, sem_al, sem_vl):
    cp_tl = pltpu.make_async_copy(tl_hbm, tl_v, sem_tl)
    cp_al = pltpu.make_async_copy(al_hbm, al_v, sem_al)
    cp_vl = pltpu.make_async_copy(vl_hbm, vl_v, sem_vl)
    cp_tl.start()
    cp_al.start()
    cp_vl.start()

    s = 0.5 / (3.0 * DELTA)

    feats_all = (tf_ref[:], af_ref[:], vf_ref[:])
    f2_all = tuple(jnp.sum(f * f, axis=1) for f in feats_all)

    fused = fr_ref[:]
    lcent = lc_ref[:]
    logits = jnp.dot(fused, lcent.T,
                     preferred_element_type=jnp.float32)
    pred_f = jnp.argmax(logits, axis=1)
    onehot_f = (pred_f[:, None] == jnp.arange(C)[None, :]).astype(jnp.float32)
    fu2 = jnp.sum(fused * fused, axis=1)

    nf_ref[0:B, :] = fused
    nf_ref[B:B + C, :] = lcent

    wacc = jnp.zeros((B, B), dtype=jnp.float32)
    centers_sum = jnp.zeros((C, D), dtype=jnp.float32)
    for cp, l_v, feats, f2 in zip((cp_tl, cp_al, cp_vl), (tl_v, al_v, vl_v),
                                  feats_all, f2_all):
        cp.wait()
        labels = l_v[:]

        pred = jnp.argmax(labels, axis=1)
        onehot = (pred[:, None] == jnp.arange(C)[None, :]).astype(jnp.float32)

        logp = jnp.log(labels)
        h = jnp.sum(labels * logp, axis=1, keepdims=True)
        ones = jnp.ones((B, 1), dtype=jnp.float32)
        U = jnp.concatenate([labels, logp, h, ones], axis=1)
        V = jnp.concatenate([s * logp, s * labels, -s * ones,
                             1.0 / 3.0 - s * h], axis=1)
        T = jnp.dot(U, V.T, preferred_element_type=jnp.float32)

        counts = jnp.sum(onehot, axis=0)
        centers = jnp.dot(onehot.T, feats,
                          preferred_element_type=jnp.float32)
        centers = centers / jnp.maximum(counts, 1.0)[:, None]
        centers_sum = centers_sum + centers

        c2 = jnp.sum(centers * centers, axis=1)
        G = jnp.dot(feats, centers.T,
                    preferred_element_type=jnp.float32)
        g = jnp.exp(-0.5 * (f2[:, None] + c2[None, :] - 2.0 * G))

        dij = jnp.dot(g, onehot.T, preferred_element_type=jnp.float32)
        dji = jnp.dot(onehot, g.T, preferred_element_type=jnp.float32)

        wacc = wacc + jnp.maximum(T, 0.0) * dij * dji

    ri = jax.lax.broadcasted_iota(jnp.int32, (B, B), 0)
    ci = jax.lax.broadcasted_iota(jnp.int32, (B, B), 1)
    w = jnp.where(ri == ci, 0.0, wacc)

    avg_c = centers_sum * (1.0 / 3.0)
    a2 = jnp.sum(avg_c * avg_c, axis=1)
    Gf = jnp.dot(fused, avg_c.T,
                 preferred_element_type=jnp.float32)
    d2f = fu2 + jnp.sum(onehot_f * (a2[None, :] - 2.0 * Gf), axis=1)
    wf = jnp.exp(-0.5 * d2f)

    R = wf[:, None] * onehot_f

    adj_ref[0:B, 0:B] = w
    adj_ref[0:B, B:B + C] = R
    adj_ref[B:B + C, 0:B] = R.T
    adj_ref[B:B + C, B:B + C] = jnp.zeros((C, C), dtype=jnp.float32)


@functools.partial(jax.jit)
def kernel(text_features, audio_features, vision_features, text_labels,
           audio_labels, vision_labels, fused_representations,
           learnable_class_centers):
    n = B + C
    any_spec = pl.BlockSpec(memory_space=pl.ANY)
    vmem_spec = pl.BlockSpec(memory_space=pltpu.VMEM)
    adj, node_features = pl.pallas_call(
        _fused_kernel,
        in_specs=[vmem_spec, vmem_spec, vmem_spec,
                  any_spec, any_spec, any_spec,
                  vmem_spec, vmem_spec],
        out_shape=(
            jax.ShapeDtypeStruct((n, n), jnp.float32),
            jax.ShapeDtypeStruct((n, D), jnp.float32),
        ),
        scratch_shapes=[
            pltpu.VMEM((B, C), jnp.float32),
            pltpu.VMEM((B, C), jnp.float32),
            pltpu.VMEM((B, C), jnp.float32),
            pltpu.SemaphoreType.DMA,
            pltpu.SemaphoreType.DMA,
            pltpu.SemaphoreType.DMA,
        ],
    )(text_features, audio_features, vision_features,
      text_labels, audio_labels, vision_labels,
      fused_representations, learnable_class_centers)
    return adj, node_features


# speedup vs baseline: 3.7951x; 1.5518x over previous
import functools

import jax
import jax.numpy as jnp
from jax.experimental import pallas as pl
from jax.experimental.pallas import tpu as pltpu

B = 512
D = 512
C = 3
DELTA = 1.5


def _row_argmax_onehot(lT):
    l0, l1, l2 = lT[0], lT[1], lT[2]
    is0 = jnp.logical_and(l0 >= l1, l0 >= l2)
    is1 = jnp.logical_and(jnp.logical_not(is0), l1 >= l2)
    is2 = jnp.logical_not(jnp.logical_or(is0, is1))
    return jnp.stack([is0.astype(jnp.float32),
                      is1.astype(jnp.float32),
                      is2.astype(jnp.float32)], axis=0)


def _fused_kernel(tf_ref, af_ref, vf_ref, labsT_ref, fr_ref, lc_ref,
                  adj_ref, nf_ref):
    s = 0.5 / (3.0 * DELTA)

    wacc = jnp.zeros((B, B), dtype=jnp.float32)
    centers_sum = jnp.zeros((C, D), dtype=jnp.float32)

    for m, f_ref in enumerate((tf_ref, af_ref, vf_ref)):
        feats = f_ref[:]
        labT = labsT_ref[C * m:C * m + C, :]

        onehotT = _row_argmax_onehot(labT)

        logpT = jnp.log(labT)
        hT = jnp.sum(labT * logpT, axis=0, keepdims=True)
        ones = jnp.ones((1, B), dtype=jnp.float32)
        UT = jnp.concatenate([labT, logpT, hT, ones], axis=0)
        VT = jnp.concatenate([s * logpT, s * labT, -s * ones,
                              1.0 / 3.0 - s * hT], axis=0)
        T = jax.lax.dot_general(UT, VT, (((0,), (0,)), ((), ())),
                                preferred_element_type=jnp.float32)

        counts = jnp.sum(onehotT, axis=1)
        centers = jnp.dot(onehotT, feats,
                          preferred_element_type=jnp.float32)
        centers = centers / jnp.maximum(counts, 1.0)[:, None]
        centers_sum = centers_sum + centers

        f2 = jnp.sum(feats * feats, axis=1)
        c2 = jnp.sum(centers * centers, axis=1)
        GT = jax.lax.dot_general(centers, feats, (((1,), (1,)), ((), ())),
                                 preferred_element_type=jnp.float32)
        gT = jnp.exp(-0.5 * (f2[None, :] + c2[:, None] - 2.0 * GT))

        dij = jax.lax.dot_general(gT, onehotT, (((0,), (0,)), ((), ())),
                                  preferred_element_type=jnp.float32)
        dji = jax.lax.dot_general(onehotT, gT, (((0,), (0,)), ((), ())),
                                  preferred_element_type=jnp.float32)

        wacc = wacc + jnp.maximum(T, 0.0) * dij * dji

    ri = jax.lax.broadcasted_iota(jnp.int32, (B, B), 0)
    ci = jax.lax.broadcasted_iota(jnp.int32, (B, B), 1)
    w = jnp.where(ri == ci, 0.0, wacc)

    fused = fr_ref[:]
    lcent = lc_ref[:]
    logitsT = jax.lax.dot_general(lcent, fused, (((1,), (1,)), ((), ())),
                                  preferred_element_type=jnp.float32)
    onehot_fT = _row_argmax_onehot(logitsT)

    avg_c = centers_sum * (1.0 / 3.0)
    fu2 = jnp.sum(fused * fused, axis=1)
    a2 = jnp.sum(avg_c * avg_c, axis=1)
    GfT = jax.lax.dot_general(avg_c, fused, (((1,), (1,)), ((), ())),
                              preferred_element_type=jnp.float32)
    d2f = fu2 + jnp.sum(onehot_fT * (a2[:, None] - 2.0 * GfT), axis=0)
    wf = jnp.exp(-0.5 * d2f)

    RT = wf[None, :] * onehot_fT

    adj_ref[0:B, 0:B] = w
    adj_ref[0:B, B:B + C] = RT.T
    adj_ref[B:B + C, 0:B] = RT
    adj_ref[B:B + C, B:B + C] = jnp.zeros((C, C), dtype=jnp.float32)

    nf_ref[0:B, :] = fused
    nf_ref[B:B + C, :] = lcent


@functools.partial(jax.jit)
def kernel(text_features, audio_features, vision_features, text_labels,
           audio_labels, vision_labels, fused_representations,
           learnable_class_centers):
    n = B + C
    labsT = jnp.concatenate(
        [text_labels.T, audio_labels.T, vision_labels.T], axis=0)
    adj, node_features = pl.pallas_call(
        _fused_kernel,
        out_shape=(
            jax.ShapeDtypeStruct((n, n), jnp.float32),
            jax.ShapeDtypeStruct((n, D), jnp.float32),
        ),
    )(text_features, audio_features, vision_features, labsT,
      fused_representations, learnable_class_centers)
    return adj, node_features


# speedup vs baseline: 4.1484x; 1.0931x over previous
import functools

import jax
import jax.numpy as jnp
from jax.experimental import pallas as pl
from jax.experimental.pallas import tpu as pltpu

B = 512
D = 512
C = 3
DELTA = 1.5


def _row_argmax_onehot(lT):
    l0, l1, l2 = lT[0], lT[1], lT[2]
    is0 = jnp.logical_and(l0 >= l1, l0 >= l2)
    is1 = jnp.logical_and(jnp.logical_not(is0), l1 >= l2)
    pred = jnp.where(is0, 0, jnp.where(is1, 1, 2))
    kiota = jax.lax.broadcasted_iota(jnp.int32, (C, lT.shape[1]), 0)
    return (kiota == pred[None, :]).astype(jnp.float32)


def _fused_kernel(tf_ref, af_ref, vf_ref, labsT_ref, fr_ref, lc_ref,
                  adj_ref, nf_ref):
    s = 0.5 / (3.0 * DELTA)

    wacc = jnp.zeros((B, B), dtype=jnp.float32)
    centers_sum = jnp.zeros((C, D), dtype=jnp.float32)

    for m, f_ref in enumerate((tf_ref, af_ref, vf_ref)):
        feats = f_ref[:]
        labT = labsT_ref[C * m:C * m + C, :]

        onehotT = _row_argmax_onehot(labT)

        logpT = jnp.log(labT)
        hT = jnp.sum(labT * logpT, axis=0, keepdims=True)
        ones = jnp.ones((1, B), dtype=jnp.float32)
        UT = jnp.concatenate([labT, logpT, hT, ones], axis=0)
        VT = jnp.concatenate([s * logpT, s * labT, -s * ones,
                              1.0 / 3.0 - s * hT], axis=0)
        T = jax.lax.dot_general(UT, VT, (((0,), (0,)), ((), ())),
                                preferred_element_type=jnp.float32)

        counts = jnp.sum(onehotT, axis=1)
        centers = jnp.dot(onehotT, feats,
                          preferred_element_type=jnp.float32)
        centers = centers / jnp.maximum(counts, 1.0)[:, None]
        centers_sum = centers_sum + centers

        f2k = jnp.sum(feats * feats, axis=1, keepdims=True)
        c2k = jnp.sum(centers * centers, axis=1, keepdims=True)
        onesC = jnp.ones((C, 1), dtype=jnp.float32)
        onesB = jnp.ones((B, 1), dtype=jnp.float32)
        lhs2 = jnp.concatenate([-0.5 * c2k, onesC], axis=1)
        rhs2 = jnp.concatenate([onesB, -0.5 * f2k], axis=1)
        GT = jax.lax.dot_general(centers, feats, (((1,), (1,)), ((), ())),
                                 preferred_element_type=jnp.float32)
        GT = GT + jax.lax.dot_general(lhs2, rhs2, (((1,), (1,)), ((), ())),
                                      preferred_element_type=jnp.float32)
        gT = jnp.exp(GT)

        dij = jax.lax.dot_general(gT, onehotT, (((0,), (0,)), ((), ())),
                                  preferred_element_type=jnp.float32)
        dji = jax.lax.dot_general(onehotT, gT, (((0,), (0,)), ((), ())),
                                  preferred_element_type=jnp.float32)

        wacc = wacc + jnp.maximum(T, 0.0) * dij * dji

    ri = jax.lax.broadcasted_iota(jnp.int32, (B, B), 0)
    ci = jax.lax.broadcasted_iota(jnp.int32, (B, B), 1)
    w = jnp.where(ri == ci, 0.0, wacc)

    fused = fr_ref[:]
    lcent = lc_ref[:]
    logitsT = jax.lax.dot_general(lcent, fused, (((1,), (1,)), ((), ())),
                                  preferred_element_type=jnp.float32)
    onehot_fT = _row_argmax_onehot(logitsT)

    avg_c = centers_sum * (1.0 / 3.0)
    fu2 = jnp.sum(fused * fused, axis=1)
    a2k = jnp.sum(avg_c * avg_c, axis=1, keepdims=True)
    onesBf = jnp.ones((B, 1), dtype=jnp.float32)
    GfT = jax.lax.dot_general(avg_c, fused, (((1,), (1,)), ((), ())),
                              preferred_element_type=jnp.float32)
    A2T = jax.lax.dot_general(a2k, onesBf, (((1,), (1,)), ((), ())),
                              preferred_element_type=jnp.float32)
    d2f = fu2 + jnp.sum(onehot_fT * (A2T - 2.0 * GfT), axis=0)
    wf = jnp.exp(-0.5 * d2f)

    RT = wf[None, :] * onehot_fT

    adj_ref[0:B, 0:B] = w
    adj_ref[0:B, B:B + C] = RT.T
    adj_ref[B:B + C, 0:B] = RT
    adj_ref[B:B + C, B:B + C] = jnp.zeros((C, C), dtype=jnp.float32)

    nf_ref[0:B, :] = fused
    nf_ref[B:B + C, :] = lcent


@functools.partial(jax.jit)
def kernel(text_features, audio_features, vision_features, text_labels,
           audio_labels, vision_labels, fused_representations,
           learnable_class_centers):
    n = B + C
    labsT = jnp.concatenate(
        [text_labels.T, audio_labels.T, vision_labels.T], axis=0)
    adj, node_features = pl.pallas_call(
        _fused_kernel,
        out_shape=(
            jax.ShapeDtypeStruct((n, n), jnp.float32),
            jax.ShapeDtypeStruct((n, D), jnp.float32),
        ),
    )(text_features, audio_features, vision_features, labsT,
      fused_representations, learnable_class_centers)
    return adj, node_features
